# Initial kernel scaffold; baseline (speedup 1.0000x reference)
import jax
import jax.numpy as jnp
from jax.experimental import pallas as pl


def kernel(doc_reps, classifier_w, classifier_b, doc_weights):
    raise NotImplementedError("write your pallas kernel here")


# speedup vs baseline: 1.1701x; 1.1701x over previous
import jax
import jax.numpy as jnp
from jax.experimental import pallas as pl
from jax.experimental.pallas import tpu as pltpu


def _pick_b_tile(B, prefer=256):
    upper = min(prefer, B // 2)
    for t in range(upper, 7, -1):
        if B % t == 0 and t % 8 == 0:
            return t
    return B


def _agg_classify_kernel(reps_ref, w_ref, wc_ref, bc_ref, out_ref):
    reps = reps_ref[...]
    w = w_ref[...]
    s = jnp.sum(reps * w, axis=0)
    acc = jnp.dot(s, wc_ref[...], preferred_element_type=jnp.float32)
    out_ref[...] = acc / jnp.sum(w, axis=0) + bc_ref[...]


def kernel(doc_reps, classifier_w, classifier_b, doc_weights):
    K, B, H = doc_reps.shape
    L = classifier_w.shape[1]
    b_tile = _pick_b_tile(B)

    reps = doc_reps.astype(jnp.float32)
    w = doc_weights.astype(jnp.float32).reshape(K, B, 1)
    wc = classifier_w.astype(jnp.float32)
    bc = classifier_b.astype(jnp.float32).reshape(1, L)

    out = pl.pallas_call(
        _agg_classify_kernel,
        out_shape=jax.ShapeDtypeStruct((B, L), jnp.float32),
        grid=(B // b_tile,),
        in_specs=[
            pl.BlockSpec((K, b_tile, H), lambda b: (0, b, 0)),
            pl.BlockSpec((K, b_tile, 1), lambda b: (0, b, 0)),
            pl.BlockSpec((H, L), lambda b: (0, 0)),
            pl.BlockSpec((1, L), lambda b: (0, 0)),
        ],
        out_specs=pl.BlockSpec((b_tile, L), lambda b: (b, 0)),
        compiler_params=pltpu.CompilerParams(
            dimension_semantics=("parallel",)),
    )(reps, w, wc, bc)
    return out


# speedup vs baseline: 1.2405x; 1.0602x over previous
import jax
import jax.numpy as jnp
from jax.experimental import pallas as pl
from jax.experimental.pallas import tpu as pltpu


def _pick_b_tile(B, prefer=256):
    upper = min(prefer, B // 2)
    for t in range(upper, 7, -1):
        if B % t == 0 and t % 8 == 0:
            return t
    return B


def _agg_classify_kernel(reps_ref, w_ref, wc_ref, bc_ref, out_ref):
    reps = reps_ref[...]
    w = w_ref[...]
    s = jnp.sum(reps * w, axis=0)
    acc = jnp.dot(s, wc_ref[...], preferred_element_type=jnp.float32)
    out_ref[...] = acc / jnp.sum(w, axis=0) + bc_ref[...]


def kernel(doc_reps, classifier_w, classifier_b, doc_weights):
    K, B, H = doc_reps.shape
    L = classifier_w.shape[1]
    b_tile = _pick_b_tile(B, prefer=512)

    reps = doc_reps.astype(jnp.float32)
    w = doc_weights.astype(jnp.float32).reshape(K, B, 1)
    wc = classifier_w.astype(jnp.float32)
    bc = classifier_b.astype(jnp.float32).reshape(1, L)

    out = pl.pallas_call(
        _agg_classify_kernel,
        out_shape=jax.ShapeDtypeStruct((B, L), jnp.float32),
        grid=(B // b_tile,),
        in_specs=[
            pl.BlockSpec((K, b_tile, H), lambda b: (0, b, 0)),
            pl.BlockSpec((K, b_tile, 1), lambda b: (0, b, 0)),
            pl.BlockSpec((H, L), lambda b: (0, 0)),
            pl.BlockSpec((1, L), lambda b: (0, 0)),
        ],
        out_specs=pl.BlockSpec((b_tile, L), lambda b: (b, 0)),
        compiler_params=pltpu.CompilerParams(
            dimension_semantics=("parallel",)),
    )(reps, w, wc, bc)
    return out
